# baseline (speedup 1.0000x reference)
import functools
import jax
import jax.numpy as jnp
from jax import lax
from jax.experimental import pallas as pl
from jax.experimental.pallas import tpu as pltpu, tpu_sc as plsc

_B = 16384
_D = 32
_NC = 2
_NS = 16
_NW = _NC * _NS
_CHUNK = _B // _NW
_BLK = 2048

_mesh = plsc.VectorSubcoreMesh(core_axis_name="c", subcore_axis_name="s")


@functools.partial(
    pl.kernel,
    out_type=jax.ShapeDtypeStruct((_B, _D), jnp.float32),
    mesh=_mesh,
    scratch_types=[
        pltpu.VMEM((_CHUNK,), jnp.int32),
        pltpu.VMEM((_CHUNK,), jnp.int32),
        pltpu.VMEM((36 * _D,), jnp.float32),
        pltpu.VMEM((6, _D), jnp.float32),
        pltpu.VMEM((6, _D), jnp.float32),
        pltpu.VMEM((_D,), jnp.float32),
        pltpu.VMEM((_D,), jnp.float32),
        pltpu.VMEM((_CHUNK, _D), jnp.float32),
    ],
)
def _sc_gather(col_hbm, row_hbm, ce_hbm, re_hbm, bw_hbm, bp_hbm, out_hbm,
               colv, rowv, tabv, cev, rev, bwv, bpv, outv):
    wid = lax.axis_index("s") * _NC + lax.axis_index("c")
    base = wid * _CHUNK
    pltpu.sync_copy(col_hbm.at[pl.ds(base, _CHUNK)], colv)
    pltpu.sync_copy(row_hbm.at[pl.ds(base, _CHUNK)], rowv)
    pltpu.sync_copy(ce_hbm, cev)
    pltpu.sync_copy(re_hbm, rev)
    pltpu.sync_copy(bw_hbm, bwv)
    pltpu.sync_copy(bp_hbm, bpv)

    bias0 = bwv[pl.ds(0, 16)] + bpv[pl.ds(0, 16)]
    bias1 = bwv[pl.ds(16, 16)] + bpv[pl.ds(16, 16)]

    for c in range(6):
        ce0 = cev[c, pl.ds(0, 16)] + bias0
        ce1 = cev[c, pl.ds(16, 16)] + bias1
        for r in range(6):
            tabv[pl.ds((c * 6 + r) * _D, 16)] = ce0 + rev[r, pl.ds(0, 16)]
            tabv[pl.ds((c * 6 + r) * _D + 16, 16)] = ce1 + rev[r, pl.ds(16, 16)]

    @plsc.parallel_loop(0, _CHUNK // 16, 1, unroll=2)
    def group(g):
        c16 = colv[pl.ds(g * 16, 16)]
        r16 = rowv[pl.ds(g * 16, 16)]
        cb16 = (c16 * 6 + r16) * _D
        for i in range(16):
            e = g * 16 + i
            cb = cb16[i]
            outv[e, pl.ds(0, 16)] = tabv[pl.ds(cb, 16)]
            outv[e, pl.ds(16, 16)] = tabv[pl.ds(cb + 16, 16)]

    pltpu.sync_copy(outv, out_hbm.at[pl.ds(base, _CHUNK), :])


def _tc_body(g_ref, w_ref, port_ref, ww_ref, wp_ref, out_ref):
    dn = (((1,), (1,)), ((), ()))
    acc = jax.lax.dot_general(w_ref[:], ww_ref[:], dn,
                              preferred_element_type=jnp.float32)
    acc = acc + jax.lax.dot_general(port_ref[:], wp_ref[:], dn,
                                    preferred_element_type=jnp.float32)
    out_ref[:] = acc + g_ref[:]


def kernel(col, row, weight, port, col_embed, row_embed, W_weight, b_weight, W_port, b_port):
    g = _sc_gather(col.astype(jnp.int32), row.astype(jnp.int32),
                   col_embed, row_embed, b_weight, b_port)
    grid = _B // _BLK
    return pl.pallas_call(
        _tc_body,
        grid=(grid,),
        in_specs=[
            pl.BlockSpec((_BLK, _D), lambda i: (i, 0)),
            pl.BlockSpec((_BLK, 1), lambda i: (i, 0)),
            pl.BlockSpec((_BLK, 6), lambda i: (i, 0)),
            pl.BlockSpec((_D, 1), lambda i: (0, 0)),
            pl.BlockSpec((_D, 6), lambda i: (0, 0)),
        ],
        out_specs=pl.BlockSpec((_BLK, _D), lambda i: (i, 0)),
        out_shape=jax.ShapeDtypeStruct((_B, _D), jnp.float32),
    )(g, weight, port, W_weight, W_port)


# speedup vs baseline: 1.0228x; 1.0228x over previous
import functools
import jax
import jax.numpy as jnp
from jax import lax
from jax.experimental import pallas as pl
from jax.experimental.pallas import tpu as pltpu, tpu_sc as plsc

_B = 16384
_D = 32
_NC = 2
_NS = 16
_NW = _NC * _NS
_CHUNK = _B // _NW
_BLK = 4096

_mesh = plsc.VectorSubcoreMesh(core_axis_name="c", subcore_axis_name="s")


@functools.partial(
    pl.kernel,
    out_type=jax.ShapeDtypeStruct((_B, _D), jnp.float32),
    mesh=_mesh,
    scratch_types=[
        pltpu.VMEM((_CHUNK,), jnp.int32),
        pltpu.VMEM((_CHUNK,), jnp.int32),
        pltpu.VMEM((36 * _D,), jnp.float32),
        pltpu.VMEM((6, _D), jnp.float32),
        pltpu.VMEM((6, _D), jnp.float32),
        pltpu.VMEM((_D,), jnp.float32),
        pltpu.VMEM((_D,), jnp.float32),
        pltpu.VMEM((_CHUNK, _D), jnp.float32),
    ],
)
def _sc_gather(col_hbm, row_hbm, ce_hbm, re_hbm, bw_hbm, bp_hbm, out_hbm,
               colv, rowv, tabv, cev, rev, bwv, bpv, outv):
    wid = lax.axis_index("s") * _NC + lax.axis_index("c")
    base = wid * _CHUNK
    pltpu.sync_copy(col_hbm.at[pl.ds(base, _CHUNK)], colv)
    pltpu.sync_copy(row_hbm.at[pl.ds(base, _CHUNK)], rowv)
    pltpu.sync_copy(ce_hbm, cev)
    pltpu.sync_copy(re_hbm, rev)
    pltpu.sync_copy(bw_hbm, bwv)
    pltpu.sync_copy(bp_hbm, bpv)

    bias0 = bwv[pl.ds(0, 16)] + bpv[pl.ds(0, 16)]
    bias1 = bwv[pl.ds(16, 16)] + bpv[pl.ds(16, 16)]

    for c in range(6):
        ce0 = cev[c, pl.ds(0, 16)] + bias0
        ce1 = cev[c, pl.ds(16, 16)] + bias1
        for r in range(6):
            tabv[pl.ds((c * 6 + r) * _D, 16)] = ce0 + rev[r, pl.ds(0, 16)]
            tabv[pl.ds((c * 6 + r) * _D + 16, 16)] = ce1 + rev[r, pl.ds(16, 16)]

    @plsc.parallel_loop(0, _CHUNK // 16, 1, unroll=2)
    def group(g):
        c16 = colv[pl.ds(g * 16, 16)]
        r16 = rowv[pl.ds(g * 16, 16)]
        cb16 = (c16 * 6 + r16) * _D
        for i in range(16):
            e = g * 16 + i
            cb = cb16[i]
            outv[e, pl.ds(0, 16)] = tabv[pl.ds(cb, 16)]
            outv[e, pl.ds(16, 16)] = tabv[pl.ds(cb + 16, 16)]

    pltpu.sync_copy(outv, out_hbm.at[pl.ds(base, _CHUNK), :])


def _tc_body(g_ref, w_ref, port_ref, ww_ref, wp_ref, out_ref):
    dn = (((1,), (1,)), ((), ()))
    acc = jax.lax.dot_general(w_ref[:], ww_ref[:], dn,
                              preferred_element_type=jnp.float32)
    acc = acc + jax.lax.dot_general(port_ref[:], wp_ref[:], dn,
                                    preferred_element_type=jnp.float32)
    out_ref[:] = acc + g_ref[:]


def kernel(col, row, weight, port, col_embed, row_embed, W_weight, b_weight, W_port, b_port):
    g = _sc_gather(col.astype(jnp.int32), row.astype(jnp.int32),
                   col_embed, row_embed, b_weight, b_port)
    grid = _B // _BLK
    return pl.pallas_call(
        _tc_body,
        grid=(grid,),
        in_specs=[
            pl.BlockSpec((_BLK, _D), lambda i: (i, 0)),
            pl.BlockSpec((_BLK, 1), lambda i: (i, 0)),
            pl.BlockSpec((_BLK, 6), lambda i: (i, 0)),
            pl.BlockSpec((_D, 1), lambda i: (0, 0)),
            pl.BlockSpec((_D, 6), lambda i: (0, 0)),
        ],
        out_specs=pl.BlockSpec((_BLK, _D), lambda i: (i, 0)),
        out_shape=jax.ShapeDtypeStruct((_B, _D), jnp.float32),
    )(g, weight, port, W_weight, W_port)


# speedup vs baseline: 1.0508x; 1.0274x over previous
import functools
import jax
import jax.numpy as jnp
from jax import lax
from jax.experimental import pallas as pl
from jax.experimental.pallas import tpu as pltpu, tpu_sc as plsc

_B = 16384
_D = 32
_NC = 2
_NS = 16
_NW = _NC * _NS
_CHUNK = _B // _NW

_mesh = plsc.VectorSubcoreMesh(core_axis_name="c", subcore_axis_name="s")


@functools.partial(
    pl.kernel,
    out_type=jax.ShapeDtypeStruct((_B, _D), jnp.float32),
    mesh=_mesh,
    scratch_types=[
        pltpu.VMEM((_CHUNK,), jnp.int32),
        pltpu.VMEM((_CHUNK,), jnp.int32),
        pltpu.VMEM((_CHUNK,), jnp.float32),
        pltpu.VMEM((6 * _CHUNK,), jnp.float32),
        pltpu.VMEM((36 * _D,), jnp.float32),
        pltpu.VMEM((6, _D), jnp.float32),
        pltpu.VMEM((6, _D), jnp.float32),
        pltpu.VMEM((_D,), jnp.float32),
        pltpu.VMEM((6 * _D,), jnp.float32),
        pltpu.VMEM((_D,), jnp.float32),
        pltpu.VMEM((_D,), jnp.float32),
        pltpu.VMEM((_CHUNK, _D), jnp.float32),
    ],
)
def _sc_kernel(col_hbm, row_hbm, w_hbm, port_hbm, ce_hbm, re_hbm, ww_hbm,
               bw_hbm, wp_hbm, bp_hbm, out_hbm,
               colv, rowv, wv, portv, tabv, cev, rev, wwv, wpv, bwv, bpv, outv):
    wid = lax.axis_index("s") * _NC + lax.axis_index("c")
    base = wid * _CHUNK
    pltpu.sync_copy(col_hbm.at[pl.ds(base, _CHUNK)], colv)
    pltpu.sync_copy(row_hbm.at[pl.ds(base, _CHUNK)], rowv)
    pltpu.sync_copy(w_hbm.at[pl.ds(base, _CHUNK)], wv)
    pltpu.sync_copy(port_hbm.at[pl.ds(base * 6, _CHUNK * 6)], portv)
    pltpu.sync_copy(ce_hbm, cev)
    pltpu.sync_copy(re_hbm, rev)
    pltpu.sync_copy(ww_hbm, wwv)
    pltpu.sync_copy(wp_hbm, wpv)
    pltpu.sync_copy(bw_hbm, bwv)
    pltpu.sync_copy(bp_hbm, bpv)

    wv0 = wwv[pl.ds(0, 16)]
    wv1 = wwv[pl.ds(16, 16)]
    wp = [wpv[pl.ds(j * _D + h * 16, 16)] for j in range(6) for h in range(2)]
    bias0 = bwv[pl.ds(0, 16)] + bpv[pl.ds(0, 16)]
    bias1 = bwv[pl.ds(16, 16)] + bpv[pl.ds(16, 16)]

    for c in range(6):
        ce0 = cev[c, pl.ds(0, 16)] + bias0
        ce1 = cev[c, pl.ds(16, 16)] + bias1
        for r in range(6):
            tabv[pl.ds((c * 6 + r) * _D, 16)] = ce0 + rev[r, pl.ds(0, 16)]
            tabv[pl.ds((c * 6 + r) * _D + 16, 16)] = ce1 + rev[r, pl.ds(16, 16)]

    @plsc.parallel_loop(0, _CHUNK // 16, 1, unroll=2)
    def group(g):
        c16 = colv[pl.ds(g * 16, 16)]
        r16 = rowv[pl.ds(g * 16, 16)]
        w16 = wv[pl.ds(g * 16, 16)]
        cb16 = (c16 * 6 + r16) * _D
        q = [portv[pl.ds(g * 96 + k * 16, 16)] for k in range(6)]
        for i in range(16):
            e = g * 16 + i
            cb = cb16[i]
            w = w16[i]
            p = [q[(6 * i + j) // 16][(6 * i + j) % 16] for j in range(6)]
            t0 = tabv[pl.ds(cb, 16)]
            t1 = tabv[pl.ds(cb + 16, 16)]
            a0 = ((t0 + w * wv0) + (p[0] * wp[0] + p[1] * wp[2])) + (
                (p[2] * wp[4] + p[3] * wp[6]) + (p[4] * wp[8] + p[5] * wp[10]))
            a1 = ((t1 + w * wv1) + (p[0] * wp[1] + p[1] * wp[3])) + (
                (p[2] * wp[5] + p[3] * wp[7]) + (p[4] * wp[9] + p[5] * wp[11]))
            outv[e, pl.ds(0, 16)] = a0
            outv[e, pl.ds(16, 16)] = a1

    pltpu.sync_copy(outv, out_hbm.at[pl.ds(base, _CHUNK), :])


def kernel(col, row, weight, port, col_embed, row_embed, W_weight, b_weight, W_port, b_port):
    return _sc_kernel(col.astype(jnp.int32), row.astype(jnp.int32),
                      weight.reshape(_B), port.reshape(_B * 6), col_embed, row_embed,
                      W_weight.reshape(_D), b_weight,
                      W_port.T.reshape(6 * _D), b_port)
